# baseline (speedup 1.0000x reference)
import jax
import jax.numpy as jnp
from jax.experimental import pallas as pl
from jax.experimental.pallas import tpu as pltpu

MODEL_DIM = 768
OUT_DIM = 768
NUM_EXPERTS = 8
GATE_HIDDEN = 64
TOP_K = 2
N_TOKENS = 4096

BT = 1024


def _moe_block(x_ref, w1_ref, b1_ref, w2_ref, b2_ref, we_ref, be_ref,
               out_ref, gs_ref):
    e = pl.program_id(0)
    i = pl.program_id(1)
    rows = pl.ds(i * BT, BT)
    xb = x_ref[rows, :]

    @pl.when(e == 0)
    def _gating():
        gx = jax.lax.dot_general(
            xb, w1_ref[...], (((1,), (1,)), ((), ())),
            preferred_element_type=jnp.float32)
        gx = jnp.maximum(gx + b1_ref[...], 0.0)
        logits = jax.lax.dot_general(
            gx, w2_ref[...], (((1,), (1,)), ((), ())),
            preferred_element_type=jnp.float32)
        logits = logits + b2_ref[...]

        m = jnp.max(logits, axis=1, keepdims=True)
        ex = jnp.exp(logits - m)
        w = ex / jnp.sum(ex, axis=1, keepdims=True)

        lane = jax.lax.broadcasted_iota(jnp.int32, w.shape, 1)
        m1 = jnp.max(w, axis=1, keepdims=True)
        is1 = w == m1
        i1 = jnp.min(jnp.where(is1, lane, NUM_EXPERTS), axis=1, keepdims=True)
        mask1 = lane == i1
        w_rest = jnp.where(mask1, -1.0, w)
        m2 = jnp.max(w_rest, axis=1, keepdims=True)
        is2 = w_rest == m2
        i2 = jnp.min(jnp.where(is2, lane, NUM_EXPERTS), axis=1, keepdims=True)
        mask2 = lane == i2
        denom = m1 + m2
        gs_ref[rows, :] = (jnp.where(mask1, m1, 0.0)
                           + jnp.where(mask2, m2, 0.0)) / denom

    y = jax.lax.dot_general(
        xb.astype(jnp.bfloat16), we_ref[0].astype(jnp.bfloat16),
        (((1,), (1,)), ((), ())), preferred_element_type=jnp.float32)
    y = jnp.maximum(y + be_ref[0], 0.0)

    lane = jax.lax.broadcasted_iota(jnp.int32, (BT, NUM_EXPERTS), 1)
    ge = jnp.sum(jnp.where(lane == e, gs_ref[rows, :], 0.0), axis=1,
                 keepdims=True)
    contrib = ge * y

    @pl.when(e == 0)
    def _init():
        out_ref[rows, :] = contrib

    @pl.when(e != 0)
    def _accum():
        out_ref[rows, :] = out_ref[rows, :] + contrib


@jax.jit
def kernel(x, W1, b1, W2, b2, We, be):
    n = x.shape[0]
    grid = (NUM_EXPERTS, n // BT)
    full = lambda shape: pl.BlockSpec(shape, lambda e, i: (0,) * len(shape))
    return pl.pallas_call(
        _moe_block,
        grid=grid,
        in_specs=[
            full((n, MODEL_DIM)),
            full((GATE_HIDDEN, MODEL_DIM)),
            full((1, GATE_HIDDEN)),
            full((NUM_EXPERTS, GATE_HIDDEN)),
            full((1, NUM_EXPERTS)),
            pl.BlockSpec((1, OUT_DIM, MODEL_DIM), lambda e, i: (e, 0, 0)),
            pl.BlockSpec((1, 1, OUT_DIM), lambda e, i: (e, 0, 0)),
        ],
        out_specs=full((n, OUT_DIM)),
        out_shape=jax.ShapeDtypeStruct((n, OUT_DIM), jnp.float32),
        scratch_shapes=[pltpu.VMEM((n, NUM_EXPERTS), jnp.float32)],
        compiler_params=pltpu.CompilerParams(
            dimension_semantics=("arbitrary", "arbitrary")),
    )(x, W1, b1.reshape(1, -1), W2, b2.reshape(1, -1), We,
      be.reshape(NUM_EXPERTS, 1, OUT_DIM))


# speedup vs baseline: 1.6085x; 1.6085x over previous
import jax
import jax.numpy as jnp
from jax.experimental import pallas as pl
from jax.experimental.pallas import tpu as pltpu

MODEL_DIM = 768
OUT_DIM = 768
NUM_EXPERTS = 8
GATE_HIDDEN = 64
TOP_K = 2
N_TOKENS = 4096

BT = 512


def _moe_block(x_ref, w1_ref, b1_ref, w2_ref, b2_ref, we_ref, be_ref, out_ref):
    xb = x_ref[...]

    gx = jax.lax.dot_general(
        xb, w1_ref[...], (((1,), (1,)), ((), ())),
        preferred_element_type=jnp.float32)
    gx = jnp.maximum(gx + b1_ref[...], 0.0)
    logits = jax.lax.dot_general(
        gx, w2_ref[...], (((1,), (1,)), ((), ())),
        preferred_element_type=jnp.float32)
    logits = logits + b2_ref[...]

    m = jnp.max(logits, axis=1, keepdims=True)
    ex = jnp.exp(logits - m)
    w = ex / jnp.sum(ex, axis=1, keepdims=True)

    lane = jax.lax.broadcasted_iota(jnp.int32, w.shape, 1)
    m1 = jnp.max(w, axis=1, keepdims=True)
    is1 = w == m1
    i1 = jnp.min(jnp.where(is1, lane, NUM_EXPERTS), axis=1, keepdims=True)
    mask1 = lane == i1
    w_rest = jnp.where(mask1, -1.0, w)
    m2 = jnp.max(w_rest, axis=1, keepdims=True)
    is2 = w_rest == m2
    i2 = jnp.min(jnp.where(is2, lane, NUM_EXPERTS), axis=1, keepdims=True)
    mask2 = lane == i2
    denom = m1 + m2
    gating = (jnp.where(mask1, m1, 0.0) + jnp.where(mask2, m2, 0.0)) / denom

    xb16 = xb.astype(jnp.bfloat16)
    we2d = we_ref[...].reshape(NUM_EXPERTS * OUT_DIM, MODEL_DIM)
    y_all = jax.lax.dot_general(
        xb16, we2d.astype(jnp.bfloat16), (((1,), (1,)), ((), ())),
        preferred_element_type=jnp.float32)
    be_flat = be_ref[...].reshape(1, NUM_EXPERTS * OUT_DIM)
    y_all = jnp.maximum(y_all + be_flat, 0.0)

    acc0 = jnp.zeros((xb.shape[0], OUT_DIM), dtype=jnp.float32)
    acc1 = jnp.zeros((xb.shape[0], OUT_DIM), dtype=jnp.float32)
    for e in range(NUM_EXPERTS):
        contrib = gating[:, e][:, None] * y_all[:, e * OUT_DIM:(e + 1) * OUT_DIM]
        if e % 2 == 0:
            acc0 = acc0 + contrib
        else:
            acc1 = acc1 + contrib
    out_ref[...] = acc0 + acc1


@jax.jit
def kernel(x, W1, b1, W2, b2, We, be):
    n = x.shape[0]
    grid = (n // BT,)
    full = lambda shape: pl.BlockSpec(shape, lambda i: (0,) * len(shape))
    return pl.pallas_call(
        _moe_block,
        grid=grid,
        in_specs=[
            pl.BlockSpec((BT, MODEL_DIM), lambda i: (i, 0)),
            full((GATE_HIDDEN, MODEL_DIM)),
            full((1, GATE_HIDDEN)),
            full((NUM_EXPERTS, GATE_HIDDEN)),
            full((1, NUM_EXPERTS)),
            full((NUM_EXPERTS, OUT_DIM, MODEL_DIM)),
            full((NUM_EXPERTS, OUT_DIM)),
        ],
        out_specs=pl.BlockSpec((BT, OUT_DIM), lambda i: (i, 0)),
        out_shape=jax.ShapeDtypeStruct((n, OUT_DIM), jnp.float32),
        compiler_params=pltpu.CompilerParams(
            dimension_semantics=("parallel",)),
    )(x, W1, b1.reshape(1, -1), W2, b2.reshape(1, -1), We, be)
